# baseline (speedup 1.0000x reference)
import functools

import jax
import jax.numpy as jnp
from jax import lax
from jax.experimental import pallas as pl
from jax.experimental.pallas import tpu as pltpu
from jax.experimental.pallas import tpu_sc as plsc

N_ROWS = 320000
COLS = 5
NBLK = N_ROWS // 128
NW = 32

SC_BLKS = 1140
TC_BLKS = NBLK - SC_BLKS
SC_BASE = TC_BLKS * 128

SC_PER_TILE = SC_BLKS // NW
SC_EXTRA = SC_BLKS - SC_PER_TILE * NW
SC_CHUNK_BLKS = 7
SC_NCHUNK = SC_PER_TILE // SC_CHUNK_BLKS
SC_CW = SC_CHUNK_BLKS * 128

TC_WBLK = 136
TC_W = TC_WBLK * 128
TC_GRID = 10

_mesh = plsc.VectorSubcoreMesh(core_axis_name="c", subcore_axis_name="s")


@functools.partial(
    pl.kernel,
    mesh=_mesh,
    compiler_params=pltpu.CompilerParams(
        needs_layout_passes=False, skip_device_barrier=True),
    out_type=[
        jax.ShapeDtypeStruct((NW, 16), jnp.float32),
        jax.ShapeDtypeStruct((NW, 16), jnp.float32),
    ],
    scratch_types=[
        pltpu.VMEM((2, COLS, SC_CW), jnp.float32),
        pltpu.VMEM((2, COLS, SC_CW), jnp.float32),
        pltpu.VMEM((2, SC_CW), jnp.int32),
        pltpu.VMEM((16,), jnp.float32),
        pltpu.VMEM((16,), jnp.float32),
        pltpu.SemaphoreType.DMA,
        pltpu.SemaphoreType.DMA,
    ],
)
def _sc_partials(pred_hbm, lab_hbm, tgt_hbm, out_loss, out_cnt,
                 pred_b, tgt_b, lab_b, stage_l, stage_c, sem0, sem1):
    wid = lax.axis_index("s") * 2 + lax.axis_index("c")
    base_row = SC_BASE + wid * (SC_PER_TILE * 128)
    sems = (sem0, sem1)

    def issue(ci, b):
        rb = base_row + ci * SC_CW
        return [
            pltpu.async_copy(pred_hbm.at[:, pl.ds(rb, SC_CW)], pred_b.at[b], sems[b]),
            pltpu.async_copy(tgt_hbm.at[:, pl.ds(rb, SC_CW)], tgt_b.at[b], sems[b]),
            pltpu.async_copy(lab_hbm.at[pl.ds(rb, SC_CW)], lab_b.at[b], sems[b]),
        ]

    def vec_step(b, o, cr):
        ac, cn = cr
        lv = lab_b[b, pl.ds(o, 16)]
        m = lv == 1
        hsum = jnp.zeros((16,), jnp.float32)
        for c in range(COLS):
            p = pred_b[b, c, pl.ds(o, 16)]
            t = tgt_b[b, c, pl.ds(o, 16)]
            d = p - t
            ax = jnp.abs(d)
            mn = jnp.minimum(ax, 1.0)
            hsum = hsum + (0.5 * mn * mn + (ax - mn))
        return (ac + jnp.where(m, hsum, 0.0), cn + jnp.where(m, 1.0, 0.0))

    def chunk_body(b, nvec, carry, unroll=4):
        def jbody(j, cr):
            o = j * (16 * unroll)
            for u in range(unroll):
                cr = vec_step(b, o + u * 16, cr)
            return cr
        return lax.fori_loop(0, nvec // unroll, jbody, carry)

    acc = jnp.zeros((16,), jnp.float32)
    cnt = jnp.zeros((16,), jnp.float32)
    pending = issue(0, 0)
    for ci in range(SC_NCHUNK):
        b = ci % 2
        nxt = issue(ci + 1, 1 - b) if ci + 1 < SC_NCHUNK else None
        for h in pending:
            h.wait()
        pending = nxt
        acc, cnt = chunk_body(b, SC_CW // 16, (acc, cnt))

    stage_l[...] = acc
    stage_c[...] = cnt

    @pl.when(wid < SC_EXTRA)
    def _():
        rb = SC_BASE + (SC_BLKS - SC_EXTRA) * 128 + wid * 128
        hs = [
            pltpu.async_copy(pred_hbm.at[:, pl.ds(rb, 128)],
                             pred_b.at[0, :, pl.ds(0, 128)], sem0),
            pltpu.async_copy(tgt_hbm.at[:, pl.ds(rb, 128)],
                             tgt_b.at[0, :, pl.ds(0, 128)], sem0),
            pltpu.async_copy(lab_hbm.at[pl.ds(rb, 128)],
                             lab_b.at[0, pl.ds(0, 128)], sem0),
        ]
        for h in hs:
            h.wait()
        a1, c1 = chunk_body(0, 8, (stage_l[...], stage_c[...]))
        stage_l[...] = a1
        stage_c[...] = c1

    pltpu.sync_copy(stage_l, out_loss.at[wid])
    pltpu.sync_copy(stage_c, out_cnt.at[wid])


def _tc_body(pred_ref, lab_ref, tgt_ref, out_ref, acc_ref):
    i = pl.program_id(0)

    @pl.when(i == 0)
    def _():
        acc_ref[0] = 0.0
        acc_ref[1] = 0.0

    d = pred_ref[...] - tgt_ref[...]
    ax = jnp.abs(d)
    mn = jnp.minimum(ax, 1.0)
    h = 0.5 * mn * mn + (ax - mn)
    m = lab_ref[...] == 1
    mb = jnp.broadcast_to(m[None, :], h.shape)
    hm = jnp.where(mb, h, 0.0)
    acc_ref[0] += jnp.sum(hm)
    acc_ref[1] += jnp.sum(jnp.where(m, 1.0, 0.0))

    @pl.when(i == TC_GRID - 1)
    def _():
        out_ref[0] = acc_ref[0]
        out_ref[1] = acc_ref[1]


_tc_partials = pl.pallas_call(
    _tc_body,
    grid=(TC_GRID,),
    in_specs=[
        pl.BlockSpec((COLS, TC_W), lambda i: (0, i)),
        pl.BlockSpec((TC_W,), lambda i: (i,)),
        pl.BlockSpec((COLS, TC_W), lambda i: (0, i)),
    ],
    out_specs=pl.BlockSpec(memory_space=pltpu.SMEM),
    out_shape=jax.ShapeDtypeStruct((2,), jnp.float32),
    scratch_shapes=[pltpu.SMEM((2,), jnp.float32)],
)


def _fin_body(sc_l_ref, sc_c_ref, tc_ref, out_ref):
    loss_sum = tc_ref[0] + jnp.sum(sc_l_ref[...])
    n_pos = tc_ref[1] + jnp.sum(sc_c_ref[...])
    out_ref[0] = loss_sum / (jnp.float32(COLS) * jnp.maximum(n_pos, 1.0))


_finalize = pl.pallas_call(
    _fin_body,
    in_specs=[
        pl.BlockSpec((NW, 16), lambda: (0, 0)),
        pl.BlockSpec((NW, 16), lambda: (0, 0)),
        pl.BlockSpec(memory_space=pltpu.SMEM),
    ],
    out_specs=pl.BlockSpec(memory_space=pltpu.SMEM),
    out_shape=jax.ShapeDtypeStruct((1,), jnp.float32),
)


def kernel(out_ellipse, labels, ellipse_targets):
    pred_t = out_ellipse.T
    tgt_t = ellipse_targets.T
    lab = jnp.reshape(labels, (-1,))
    loss_p, cnt_p = _sc_partials(pred_t, lab, tgt_t)
    tc_p = _tc_partials(pred_t, lab, tgt_t)
    out = _finalize(loss_p, cnt_p, tc_p)
    return jnp.reshape(out, ())


# speedup vs baseline: 1.0317x; 1.0317x over previous
import functools

import jax
import jax.numpy as jnp
from jax import lax
from jax.experimental import pallas as pl
from jax.experimental.pallas import tpu as pltpu
from jax.experimental.pallas import tpu_sc as plsc

N_ROWS = 320000
COLS = 5
NBLK = N_ROWS // 128
NW = 32

SC_BLKS = 900
TC_BLKS = NBLK - SC_BLKS
SC_BASE = TC_BLKS * 128

SC_PER_TILE = SC_BLKS // NW
SC_EXTRA = SC_BLKS - SC_PER_TILE * NW
SC_CHUNK_BLKS = 14
SC_NCHUNK = SC_PER_TILE // SC_CHUNK_BLKS
SC_CW = SC_CHUNK_BLKS * 128

TC_WBLK = 200
TC_W = TC_WBLK * 128
TC_GRID = 8

_mesh = plsc.VectorSubcoreMesh(core_axis_name="c", subcore_axis_name="s")


@functools.partial(
    pl.kernel,
    mesh=_mesh,
    compiler_params=pltpu.CompilerParams(
        needs_layout_passes=False, skip_device_barrier=True),
    out_type=[
        jax.ShapeDtypeStruct((NW, 16), jnp.float32),
        jax.ShapeDtypeStruct((NW, 16), jnp.float32),
    ],
    scratch_types=[
        pltpu.VMEM((2, COLS, SC_CW), jnp.float32),
        pltpu.VMEM((2, COLS, SC_CW), jnp.float32),
        pltpu.VMEM((2, SC_CW), jnp.int32),
        pltpu.VMEM((16,), jnp.float32),
        pltpu.VMEM((16,), jnp.float32),
        pltpu.SemaphoreType.DMA,
        pltpu.SemaphoreType.DMA,
    ],
)
def _sc_partials(pred_hbm, lab_hbm, tgt_hbm, out_loss, out_cnt,
                 pred_b, tgt_b, lab_b, stage_l, stage_c, sem0, sem1):
    wid = lax.axis_index("s") * 2 + lax.axis_index("c")
    base_row = SC_BASE + wid * (SC_PER_TILE * 128)
    sems = (sem0, sem1)

    def issue(ci, b):
        rb = base_row + ci * SC_CW
        return [
            pltpu.async_copy(pred_hbm.at[:, pl.ds(rb, SC_CW)], pred_b.at[b], sems[b]),
            pltpu.async_copy(tgt_hbm.at[:, pl.ds(rb, SC_CW)], tgt_b.at[b], sems[b]),
            pltpu.async_copy(lab_hbm.at[pl.ds(rb, SC_CW)], lab_b.at[b], sems[b]),
        ]

    def vec_step(b, o, cr):
        ac, cn = cr
        lv = lab_b[b, pl.ds(o, 16)]
        m = lv == 1
        hsum = jnp.zeros((16,), jnp.float32)
        for c in range(COLS):
            p = pred_b[b, c, pl.ds(o, 16)]
            t = tgt_b[b, c, pl.ds(o, 16)]
            d = p - t
            ax = jnp.abs(d)
            mn = jnp.minimum(ax, 1.0)
            hsum = hsum + (0.5 * mn * mn + (ax - mn))
        return (ac + jnp.where(m, hsum, 0.0), cn + jnp.where(m, 1.0, 0.0))

    def chunk_body(b, nvec, carry, unroll=4):
        def jbody(j, cr):
            o = j * (16 * unroll)
            for u in range(unroll):
                cr = vec_step(b, o + u * 16, cr)
            return cr
        return lax.fori_loop(0, nvec // unroll, jbody, carry)

    acc = jnp.zeros((16,), jnp.float32)
    cnt = jnp.zeros((16,), jnp.float32)
    pending = issue(0, 0)
    for ci in range(SC_NCHUNK):
        b = ci % 2
        nxt = issue(ci + 1, 1 - b) if ci + 1 < SC_NCHUNK else None
        for h in pending:
            h.wait()
        pending = nxt
        acc, cnt = chunk_body(b, SC_CW // 16, (acc, cnt))

    stage_l[...] = acc
    stage_c[...] = cnt

    @pl.when(wid < SC_EXTRA)
    def _():
        rb = SC_BASE + (SC_BLKS - SC_EXTRA) * 128 + wid * 128
        hs = [
            pltpu.async_copy(pred_hbm.at[:, pl.ds(rb, 128)],
                             pred_b.at[0, :, pl.ds(0, 128)], sem0),
            pltpu.async_copy(tgt_hbm.at[:, pl.ds(rb, 128)],
                             tgt_b.at[0, :, pl.ds(0, 128)], sem0),
            pltpu.async_copy(lab_hbm.at[pl.ds(rb, 128)],
                             lab_b.at[0, pl.ds(0, 128)], sem0),
        ]
        for h in hs:
            h.wait()
        a1, c1 = chunk_body(0, 8, (stage_l[...], stage_c[...]))
        stage_l[...] = a1
        stage_c[...] = c1

    pltpu.sync_copy(stage_l, out_loss.at[wid])
    pltpu.sync_copy(stage_c, out_cnt.at[wid])


def _tc_body(pred_ref, lab_ref, tgt_ref, out_ref, acc_ref):
    i = pl.program_id(0)

    @pl.when(i == 0)
    def _():
        acc_ref[0] = 0.0
        acc_ref[1] = 0.0

    d = pred_ref[...] - tgt_ref[...]
    ax = jnp.abs(d)
    mn = jnp.minimum(ax, 1.0)
    h = 0.5 * mn * mn + (ax - mn)
    m = lab_ref[...] == 1
    mb = jnp.broadcast_to(m[None, :], h.shape)
    hm = jnp.where(mb, h, 0.0)
    acc_ref[0] += jnp.sum(hm)
    acc_ref[1] += jnp.sum(jnp.where(m, 1.0, 0.0))

    @pl.when(i == TC_GRID - 1)
    def _():
        out_ref[0] = acc_ref[0]
        out_ref[1] = acc_ref[1]


_tc_partials = pl.pallas_call(
    _tc_body,
    grid=(TC_GRID,),
    in_specs=[
        pl.BlockSpec((COLS, TC_W), lambda i: (0, i)),
        pl.BlockSpec((TC_W,), lambda i: (i,)),
        pl.BlockSpec((COLS, TC_W), lambda i: (0, i)),
    ],
    out_specs=pl.BlockSpec(memory_space=pltpu.SMEM),
    out_shape=jax.ShapeDtypeStruct((2,), jnp.float32),
    scratch_shapes=[pltpu.SMEM((2,), jnp.float32)],
)


def _fin_body(sc_l_ref, sc_c_ref, tc_ref, out_ref):
    loss_sum = tc_ref[0] + jnp.sum(sc_l_ref[...])
    n_pos = tc_ref[1] + jnp.sum(sc_c_ref[...])
    out_ref[0] = loss_sum / (jnp.float32(COLS) * jnp.maximum(n_pos, 1.0))


_finalize = pl.pallas_call(
    _fin_body,
    in_specs=[
        pl.BlockSpec((NW, 16), lambda: (0, 0)),
        pl.BlockSpec((NW, 16), lambda: (0, 0)),
        pl.BlockSpec(memory_space=pltpu.SMEM),
    ],
    out_specs=pl.BlockSpec(memory_space=pltpu.SMEM),
    out_shape=jax.ShapeDtypeStruct((1,), jnp.float32),
)


def kernel(out_ellipse, labels, ellipse_targets):
    pred_t = out_ellipse.T
    tgt_t = ellipse_targets.T
    lab = jnp.reshape(labels, (-1,))
    loss_p, cnt_p = _sc_partials(pred_t, lab, tgt_t)
    tc_p = _tc_partials(pred_t, lab, tgt_t)
    out = _finalize(loss_p, cnt_p, tc_p)
    return jnp.reshape(out, ())


# speedup vs baseline: 1.0385x; 1.0066x over previous
import functools

import jax
import jax.numpy as jnp
from jax import lax
from jax.experimental import pallas as pl
from jax.experimental.pallas import tpu as pltpu
from jax.experimental.pallas import tpu_sc as plsc

N_ROWS = 320000
COLS = 5
NBLK = N_ROWS // 128
NW = 32

SC_BLKS = 900
TC_BLKS = NBLK - SC_BLKS
SC_BASE = TC_BLKS * 128

SC_PER_TILE = SC_BLKS // NW
SC_EXTRA = SC_BLKS - SC_PER_TILE * NW
SC_CHUNK_BLKS = 14
SC_NCHUNK = SC_PER_TILE // SC_CHUNK_BLKS
SC_CW = SC_CHUNK_BLKS * 128

TC_WBLK = 320
TC_W = TC_WBLK * 128
TC_GRID = 5

_mesh = plsc.VectorSubcoreMesh(core_axis_name="c", subcore_axis_name="s")


@functools.partial(
    pl.kernel,
    mesh=_mesh,
    compiler_params=pltpu.CompilerParams(
        needs_layout_passes=False, skip_device_barrier=True),
    out_type=[
        jax.ShapeDtypeStruct((NW, 16), jnp.float32),
        jax.ShapeDtypeStruct((NW, 16), jnp.float32),
    ],
    scratch_types=[
        pltpu.VMEM((2, COLS, SC_CW), jnp.float32),
        pltpu.VMEM((2, COLS, SC_CW), jnp.float32),
        pltpu.VMEM((2, SC_CW), jnp.int32),
        pltpu.VMEM((16,), jnp.float32),
        pltpu.VMEM((16,), jnp.float32),
        pltpu.SemaphoreType.DMA,
        pltpu.SemaphoreType.DMA,
    ],
)
def _sc_partials(pred_hbm, lab_hbm, tgt_hbm, out_loss, out_cnt,
                 pred_b, tgt_b, lab_b, stage_l, stage_c, sem0, sem1):
    wid = lax.axis_index("s") * 2 + lax.axis_index("c")
    base_row = SC_BASE + wid * (SC_PER_TILE * 128)
    sems = (sem0, sem1)

    def issue(ci, b):
        rb = base_row + ci * SC_CW
        return [
            pltpu.async_copy(pred_hbm.at[:, pl.ds(rb, SC_CW)], pred_b.at[b], sems[b]),
            pltpu.async_copy(tgt_hbm.at[:, pl.ds(rb, SC_CW)], tgt_b.at[b], sems[b]),
            pltpu.async_copy(lab_hbm.at[pl.ds(rb, SC_CW)], lab_b.at[b], sems[b]),
        ]

    def vec_step(b, o, cr):
        ac, cn = cr
        lv = lab_b[b, pl.ds(o, 16)]
        m = lv == 1
        hsum = jnp.zeros((16,), jnp.float32)
        for c in range(COLS):
            p = pred_b[b, c, pl.ds(o, 16)]
            t = tgt_b[b, c, pl.ds(o, 16)]
            d = p - t
            ax = jnp.abs(d)
            mn = jnp.minimum(ax, 1.0)
            hsum = hsum + (0.5 * mn * mn + (ax - mn))
        return (ac + jnp.where(m, hsum, 0.0), cn + jnp.where(m, 1.0, 0.0))

    def chunk_body(b, nvec, carry, unroll=4):
        def jbody(j, cr):
            o = j * (16 * unroll)
            for u in range(unroll):
                cr = vec_step(b, o + u * 16, cr)
            return cr
        return lax.fori_loop(0, nvec // unroll, jbody, carry)

    acc = jnp.zeros((16,), jnp.float32)
    cnt = jnp.zeros((16,), jnp.float32)
    pending = issue(0, 0)
    for ci in range(SC_NCHUNK):
        b = ci % 2
        nxt = issue(ci + 1, 1 - b) if ci + 1 < SC_NCHUNK else None
        for h in pending:
            h.wait()
        pending = nxt
        acc, cnt = chunk_body(b, SC_CW // 16, (acc, cnt))

    stage_l[...] = acc
    stage_c[...] = cnt

    @pl.when(wid < SC_EXTRA)
    def _():
        rb = SC_BASE + (SC_BLKS - SC_EXTRA) * 128 + wid * 128
        hs = [
            pltpu.async_copy(pred_hbm.at[:, pl.ds(rb, 128)],
                             pred_b.at[0, :, pl.ds(0, 128)], sem0),
            pltpu.async_copy(tgt_hbm.at[:, pl.ds(rb, 128)],
                             tgt_b.at[0, :, pl.ds(0, 128)], sem0),
            pltpu.async_copy(lab_hbm.at[pl.ds(rb, 128)],
                             lab_b.at[0, pl.ds(0, 128)], sem0),
        ]
        for h in hs:
            h.wait()
        a1, c1 = chunk_body(0, 8, (stage_l[...], stage_c[...]))
        stage_l[...] = a1
        stage_c[...] = c1

    pltpu.sync_copy(stage_l, out_loss.at[wid])
    pltpu.sync_copy(stage_c, out_cnt.at[wid])


def _tc_body(pred_ref, lab_ref, tgt_ref, out_ref, acc_ref):
    i = pl.program_id(0)

    @pl.when(i == 0)
    def _():
        acc_ref[0] = 0.0
        acc_ref[1] = 0.0

    d = pred_ref[...] - tgt_ref[...]
    ax = jnp.abs(d)
    mn = jnp.minimum(ax, 1.0)
    h = 0.5 * mn * mn + (ax - mn)
    m = lab_ref[...] == 1
    mb = jnp.broadcast_to(m[None, :], h.shape)
    hm = jnp.where(mb, h, 0.0)
    acc_ref[0] += jnp.sum(hm)
    acc_ref[1] += jnp.sum(jnp.where(m, 1.0, 0.0))

    @pl.when(i == TC_GRID - 1)
    def _():
        out_ref[0] = acc_ref[0]
        out_ref[1] = acc_ref[1]


_tc_partials = pl.pallas_call(
    _tc_body,
    grid=(TC_GRID,),
    in_specs=[
        pl.BlockSpec((COLS, TC_W), lambda i: (0, i)),
        pl.BlockSpec((TC_W,), lambda i: (i,)),
        pl.BlockSpec((COLS, TC_W), lambda i: (0, i)),
    ],
    out_specs=pl.BlockSpec(memory_space=pltpu.SMEM),
    out_shape=jax.ShapeDtypeStruct((2,), jnp.float32),
    scratch_shapes=[pltpu.SMEM((2,), jnp.float32)],
)


def _fin_body(sc_l_ref, sc_c_ref, tc_ref, out_ref):
    loss_sum = tc_ref[0] + jnp.sum(sc_l_ref[...])
    n_pos = tc_ref[1] + jnp.sum(sc_c_ref[...])
    out_ref[0] = loss_sum / (jnp.float32(COLS) * jnp.maximum(n_pos, 1.0))


_finalize = pl.pallas_call(
    _fin_body,
    in_specs=[
        pl.BlockSpec((NW, 16), lambda: (0, 0)),
        pl.BlockSpec((NW, 16), lambda: (0, 0)),
        pl.BlockSpec(memory_space=pltpu.SMEM),
    ],
    out_specs=pl.BlockSpec(memory_space=pltpu.SMEM),
    out_shape=jax.ShapeDtypeStruct((1,), jnp.float32),
)


def kernel(out_ellipse, labels, ellipse_targets):
    pred_t = out_ellipse.T
    tgt_t = ellipse_targets.T
    lab = jnp.reshape(labels, (-1,))
    loss_p, cnt_p = _sc_partials(pred_t, lab, tgt_t)
    tc_p = _tc_partials(pred_t, lab, tgt_t)
    out = _finalize(loss_p, cnt_p, tc_p)
    return jnp.reshape(out, ())
